# Initial kernel scaffold; baseline (speedup 1.0000x reference)
import jax
import jax.numpy as jnp
from jax.experimental import pallas as pl


def kernel(x, edge_index, batch, W1, b1, W2, b2, W3, b3, W4, b4, g1, bt1, g2, bt2, g3, bt3, g4, bt4):
    raise NotImplementedError("write your pallas kernel here")


# speedup vs baseline: 1.5654x; 1.5654x over previous
import functools

import jax
import jax.numpy as jnp
from jax import lax
from jax.experimental import pallas as pl
from jax.experimental.pallas import tpu as pltpu
from jax.experimental.pallas import tpu_sc as plsc

N = 10000
E = 320000
F = 128
H = 256
G = 64

NUM_CORES = 2
NUM_TILES = 16
W = 128
CHUNK = 128
CPT = 160
NCHUNKS = NUM_TILES * CPT
EPAD = NCHUNKS * CHUNK
NHALF = 5056
NACC = 5120
ROWS_PER_TILE = NACC // NUM_TILES
EPS = 1e-5


def _prologue(srcs, dsts, zeros, src_v, dst_v, acc, s):
  pltpu.sync_copy(zeros.at[pl.ds(s * ROWS_PER_TILE, ROWS_PER_TILE)],
                  acc.at[pl.ds(s * ROWS_PER_TILE, ROWS_PER_TILE)])
  pltpu.sync_copy(srcs.at[pl.ds(s * CPT, CPT)], src_v)
  pltpu.sync_copy(dsts.at[pl.ds(s * CPT, CPT)], dst_v)


def _edge_loop(table, src_v, dst_v, rows, acc, sem):
  def step(j, carry):
    pltpu.async_copy(table.at[src_v.at[j]], rows, sem).wait()
    pltpu.sync_copy(rows, acc.at[dst_v.at[j]], add=True)
    return carry
  lax.fori_loop(0, CPT, step, 0)


def _copy_out(acc, out, s):
  pltpu.sync_copy(acc.at[pl.ds(s * ROWS_PER_TILE, ROWS_PER_TILE)],
                  out.at[pl.ds(s * ROWS_PER_TILE, ROWS_PER_TILE)])


_SC_OUT = (jax.ShapeDtypeStruct((NACC, W), jnp.float32),
           jax.ShapeDtypeStruct((NACC, W), jnp.float32))
_SC_SCRATCH = [
    pltpu.VMEM((CPT, CHUNK), jnp.int32),
    pltpu.VMEM((CPT, CHUNK), jnp.int32),
    pltpu.VMEM((CHUNK, W), jnp.float32),
    pltpu.VMEM_SHARED((NACC, W), jnp.float32),
    pltpu.SemaphoreType.DMA,
]


@functools.cache
def _make_seg_l1():
  mesh = plsc.VectorSubcoreMesh(core_axis_name="c", subcore_axis_name="s")

  def body(table, srcs, dst0, dst1, zeros, out_a, out_b,
           src_v, dst_v, rows, acc, sem):
    c = lax.axis_index("c")
    s = lax.axis_index("s")

    @pl.when(c == 0)
    def _():
      _prologue(srcs, dst0, zeros, src_v, dst_v, acc, s)

    @pl.when(c == 1)
    def _():
      _prologue(srcs, dst1, zeros, src_v, dst_v, acc, s)

    plsc.subcore_barrier()
    _edge_loop(table, src_v, dst_v, rows, acc, sem)
    plsc.subcore_barrier()

    @pl.when(c == 0)
    def _():
      _copy_out(acc, out_a, s)

    @pl.when(c == 1)
    def _():
      _copy_out(acc, out_b, s)

  return pl.kernel(body, out_type=_SC_OUT, mesh=mesh,
                   scratch_types=_SC_SCRATCH)


@functools.cache
def _make_seg_l2():
  mesh = plsc.VectorSubcoreMesh(core_axis_name="c", subcore_axis_name="s")

  def body(table_a, table_b, srcs, dsts, zeros, out_a, out_b,
           src_v, dst_v, rows, acc, sem):
    c = lax.axis_index("c")
    s = lax.axis_index("s")
    _prologue(srcs, dsts, zeros, src_v, dst_v, acc, s)
    plsc.subcore_barrier()

    @pl.when(c == 0)
    def _():
      _edge_loop(table_a, src_v, dst_v, rows, acc, sem)

    @pl.when(c == 1)
    def _():
      _edge_loop(table_b, src_v, dst_v, rows, acc, sem)

    plsc.subcore_barrier()

    @pl.when(c == 0)
    def _():
      _copy_out(acc, out_a, s)

    @pl.when(c == 1)
    def _():
      _copy_out(acc, out_b, s)

  return pl.kernel(body, out_type=_SC_OUT, mesh=mesh,
                   scratch_types=_SC_SCRATCH)


_DOT = functools.partial(jnp.dot, precision=jax.lax.Precision.HIGHEST,
                         preferred_element_type=jnp.float32)

BR = 2000
NB = N // BR

_row_spec = lambda w: pl.BlockSpec((BR, w), lambda i: (i, 0))
_full_spec = lambda r, w: pl.BlockSpec((r, w), lambda i: (0, 0))
_STATS = jax.ShapeDtypeStruct((2, H), jnp.float32)


def _accum_stats(i, vals, acc, stat_out):
  s1 = jnp.sum(vals, axis=0, keepdims=True)
  s2 = jnp.sum(vals * vals, axis=0, keepdims=True)
  part = jnp.concatenate([s1, s2], axis=0)

  @pl.when(i == 0)
  def _():
    acc[...] = part

  @pl.when(i > 0)
  def _():
    acc[...] = acc[...] + part

  @pl.when(i == NB - 1)
  def _():
    stat_out[...] = acc[...]


def _bn_from_stats(t, stats, gamma, beta):
  m = stats[0:1] * (1.0 / N)
  v = stats[1:2] * (1.0 / N) - m * m
  return (t - m) * lax.rsqrt(v + EPS) * gamma + beta


def _lin_body(x, agg, Wr, br, t_out, stat_out, acc):
  i = pl.program_id(0)
  t = _DOT(x[...] + agg[...], Wr[...]) + br[...]
  t_out[...] = t
  _accum_stats(i, t, acc, stat_out)


def _lin(x, agg, Wr, br, win):
  return pl.pallas_call(
      _lin_body,
      grid=(NB,),
      in_specs=[_row_spec(win), _row_spec(win), _full_spec(win, H),
                _full_spec(1, H)],
      out_specs=(_row_spec(H), _full_spec(2, H)),
      out_shape=(jax.ShapeDtypeStruct((N, H), jnp.float32), _STATS),
      scratch_shapes=[pltpu.VMEM((2, H), jnp.float32)],
  )(x, agg, Wr, br)


def _bnrelu_body(t, stats, g, bt, u_out, stat_out, acc):
  i = pl.program_id(0)
  u = jnp.maximum(_bn_from_stats(t[...], stats[...], g[...], bt[...]), 0.0)
  u_out[...] = u
  _accum_stats(i, u, acc, stat_out)


def _bnrelu(t, stats, g, bt):
  return pl.pallas_call(
      _bnrelu_body,
      grid=(NB,),
      in_specs=[_row_spec(H), _full_spec(2, H), _full_spec(1, H),
                _full_spec(1, H)],
      out_specs=(_row_spec(H), _full_spec(2, H)),
      out_shape=(jax.ShapeDtypeStruct((N, H), jnp.float32), _STATS),
      scratch_shapes=[pltpu.VMEM((2, H), jnp.float32)],
  )(t, stats, g, bt)


def _bnlin_body(u, stats, g, bt, Wr, br, out, stat_out, acc, *, relu_out):
  i = pl.program_id(0)
  z = _DOT(_bn_from_stats(u[...], stats[...], g[...], bt[...]), Wr[...]) + br[...]
  if relu_out:
    z = jnp.maximum(z, 0.0)
  out[...] = z
  _accum_stats(i, z, acc, stat_out)


def _bnlin(u, stats, g, bt, Wr, br, relu_out):
  return pl.pallas_call(
      functools.partial(_bnlin_body, relu_out=relu_out),
      grid=(NB,),
      in_specs=[_row_spec(H), _full_spec(2, H), _full_spec(1, H),
                _full_spec(1, H), _full_spec(H, H), _full_spec(1, H)],
      out_specs=(_row_spec(H), _full_spec(2, H)),
      out_shape=(jax.ShapeDtypeStruct((N, H), jnp.float32), _STATS),
      scratch_shapes=[pltpu.VMEM((2, H), jnp.float32)],
  )(u, stats, g, bt, Wr, br)


def _bnrelu_pool_body(t, stats, g, bt, batch, h2_out, pool_out,
                      acc_pool, acc_cnt):
  i = pl.program_id(0)
  h2 = jnp.maximum(_bn_from_stats(t[...], stats[...], g[...], bt[...]), 0.0)
  h2_out[...] = h2
  seg = lax.broadcasted_iota(jnp.int32, (BR, G), 1)
  onehot = (batch[...] == seg).astype(jnp.float32)
  part = lax.dot_general(onehot, h2, (((0,), (0,)), ((), ())),
                         precision=jax.lax.Precision.HIGHEST,
                         preferred_element_type=jnp.float32)
  cnt = jnp.sum(onehot, axis=0)[:, None]

  @pl.when(i == 0)
  def _():
    acc_pool[...] = part
    acc_cnt[...] = cnt

  @pl.when(i > 0)
  def _():
    acc_pool[...] = acc_pool[...] + part
    acc_cnt[...] = acc_cnt[...] + cnt

  @pl.when(i == NB - 1)
  def _():
    pool_out[...] = acc_pool[...] / jnp.maximum(acc_cnt[...], 1.0)


def _bnrelu_pool(t, stats, g, bt, batch):
  return pl.pallas_call(
      _bnrelu_pool_body,
      grid=(NB,),
      in_specs=[_row_spec(H), _full_spec(2, H), _full_spec(1, H),
                _full_spec(1, H), _row_spec(1)],
      out_specs=(_row_spec(H), _full_spec(G, H)),
      out_shape=(jax.ShapeDtypeStruct((N, H), jnp.float32),
                 jax.ShapeDtypeStruct((G, H), jnp.float32)),
      scratch_shapes=[pltpu.VMEM((G, H), jnp.float32),
                      pltpu.VMEM((G, 1), jnp.float32)],
  )(t, stats, g, bt, batch)


def kernel(x, edge_index, batch, W1, b1, W2, b2, W3, b3, W4, b4,
           g1, bt1, g2, bt2, g3, bt3, g4, bt4):
  ei = edge_index.astype(jnp.int32)
  src_p = jnp.concatenate(
      [ei[0], jnp.zeros((EPAD - E,), jnp.int32)]).reshape(NCHUNKS, CHUNK)
  dst_p = jnp.concatenate(
      [ei[1], jnp.full((EPAD - E,), N, jnp.int32)]).reshape(NCHUNKS, CHUNK)
  tile_id = lax.broadcasted_iota(jnp.int32, (NCHUNKS, CHUNK), 0) // CPT
  dummy = NHALF + tile_id
  dst0 = jnp.where(dst_p < NHALF, dst_p, dummy)
  dst1 = jnp.where((dst_p >= NHALF) & (dst_p < N), dst_p - NHALF, dummy)

  zeros = jnp.zeros((NACC, W), jnp.float32)
  a1lo, a1hi = _make_seg_l1()(x, src_p, dst0, dst1, zeros)
  agg1 = jnp.concatenate([a1lo[:NHALF], a1hi[:N - NHALF]], axis=0)

  t1, st1 = _lin(x, agg1, W1, b1.reshape(1, H), F)
  u, st2 = _bnrelu(t1, st1, g1.reshape(1, H), bt1.reshape(1, H))
  h, _ = _bnlin(u, st2, g2.reshape(1, H), bt2.reshape(1, H), W2,
                b2.reshape(1, H), relu_out=True)

  ha = h[:, :H // 2]
  hb = h[:, H // 2:]
  seg_l2 = _make_seg_l2()
  a2lo_a, a2lo_b = seg_l2(ha, hb, src_p, dst0, zeros)
  a2hi_a, a2hi_b = seg_l2(ha, hb, src_p, dst1, zeros)
  agg2 = jnp.concatenate(
      [jnp.concatenate([a2lo_a[:NHALF], a2lo_b[:NHALF]], axis=1),
       jnp.concatenate([a2hi_a[:N - NHALF], a2hi_b[:N - NHALF]], axis=1)],
      axis=0)

  batch2d = batch.astype(jnp.int32).reshape(N, 1)
  t2, st3 = _lin(h, agg2, W3, b3.reshape(1, H), H)
  v, st4 = _bnlin(t2, st3, g3.reshape(1, H), bt3.reshape(1, H), W4,
                  b4.reshape(1, H), relu_out=False)
  h2, x_pool = _bnrelu_pool(v, st4, g4.reshape(1, H), bt4.reshape(1, H),
                            batch2d)
  return (x_pool, h2)


# speedup vs baseline: 1.8270x; 1.1671x over previous
import functools

import jax
import jax.numpy as jnp
from jax import lax
from jax.experimental import pallas as pl
from jax.experimental.pallas import tpu as pltpu
from jax.experimental.pallas import tpu_sc as plsc

N = 10000
E = 320000
F = 128
H = 256
G = 64

NUM_CORES = 2
NUM_TILES = 16
W = 128
CHUNK = 128
CPT = 160
NCHUNKS = NUM_TILES * CPT
EPAD = NCHUNKS * CHUNK
NHALF = 5056
NACC = 5120
ROWS_PER_TILE = NACC // NUM_TILES
EPS = 1e-5


def _prologue(srcs, dsts, zeros, src_v, dst_v, acc, s):
  pltpu.sync_copy(zeros.at[pl.ds(s * ROWS_PER_TILE, ROWS_PER_TILE)],
                  acc.at[pl.ds(s * ROWS_PER_TILE, ROWS_PER_TILE)])
  pltpu.sync_copy(srcs.at[pl.ds(s * CPT, CPT)], src_v)
  pltpu.sync_copy(dsts.at[pl.ds(s * CPT, CPT)], dst_v)


def _edge_loop(table, src_v, dst_v, rows, acc, sems):
  nbuf = len(rows)
  for b in range(nbuf):
    pltpu.async_copy(table.at[src_v.at[b]], rows[b], sems[b])

  @pl.loop(0, CPT, step=nbuf)
  def _(j):
    for b in range(nbuf):
      jj = j + b
      pltpu.make_async_copy(table.at[src_v.at[jj]], rows[b], sems[b]).wait()
      pltpu.sync_copy(rows[b], acc.at[dst_v.at[jj]], add=True)

      @pl.when(jj + nbuf < CPT)
      def _():
        pltpu.async_copy(table.at[src_v.at[jj + nbuf]], rows[b], sems[b])


def _copy_out(acc, out, s):
  pltpu.sync_copy(acc.at[pl.ds(s * ROWS_PER_TILE, ROWS_PER_TILE)],
                  out.at[pl.ds(s * ROWS_PER_TILE, ROWS_PER_TILE)])


_SC_OUT = (jax.ShapeDtypeStruct((NACC, W), jnp.float32),
           jax.ShapeDtypeStruct((NACC, W), jnp.float32))
_SC_SCRATCH = [
    pltpu.VMEM((CPT, CHUNK), jnp.int32),
    pltpu.VMEM((CPT, CHUNK), jnp.int32),
    pltpu.VMEM((CHUNK, W), jnp.float32),
    pltpu.VMEM((CHUNK, W), jnp.float32),
    pltpu.VMEM_SHARED((NACC, W), jnp.float32),
    pltpu.SemaphoreType.DMA,
    pltpu.SemaphoreType.DMA,
]


@functools.cache
def _make_seg_l1():
  mesh = plsc.VectorSubcoreMesh(core_axis_name="c", subcore_axis_name="s")

  def body(table, srcs, dst0, dst1, zeros, out_a, out_b,
           src_v, dst_v, rows0, rows1, acc, sem0, sem1):
    c = lax.axis_index("c")
    s = lax.axis_index("s")

    @pl.when(c == 0)
    def _():
      _prologue(srcs, dst0, zeros, src_v, dst_v, acc, s)

    @pl.when(c == 1)
    def _():
      _prologue(srcs, dst1, zeros, src_v, dst_v, acc, s)

    plsc.subcore_barrier()
    _edge_loop(table, src_v, dst_v, (rows0, rows1), acc, (sem0, sem1))
    plsc.subcore_barrier()

    @pl.when(c == 0)
    def _():
      _copy_out(acc, out_a, s)

    @pl.when(c == 1)
    def _():
      _copy_out(acc, out_b, s)

  return pl.kernel(body, out_type=_SC_OUT, mesh=mesh,
                   scratch_types=_SC_SCRATCH)


@functools.cache
def _make_seg_l2():
  mesh = plsc.VectorSubcoreMesh(core_axis_name="c", subcore_axis_name="s")

  def body(table_a, table_b, srcs, dsts, zeros, out_a, out_b,
           src_v, dst_v, rows0, rows1, acc, sem0, sem1):
    c = lax.axis_index("c")
    s = lax.axis_index("s")
    _prologue(srcs, dsts, zeros, src_v, dst_v, acc, s)
    plsc.subcore_barrier()

    @pl.when(c == 0)
    def _():
      _edge_loop(table_a, src_v, dst_v, (rows0, rows1), acc, (sem0, sem1))

    @pl.when(c == 1)
    def _():
      _edge_loop(table_b, src_v, dst_v, (rows0, rows1), acc, (sem0, sem1))

    plsc.subcore_barrier()

    @pl.when(c == 0)
    def _():
      _copy_out(acc, out_a, s)

    @pl.when(c == 1)
    def _():
      _copy_out(acc, out_b, s)

  return pl.kernel(body, out_type=_SC_OUT, mesh=mesh,
                   scratch_types=_SC_SCRATCH)


_DOT = functools.partial(jnp.dot, precision=jax.lax.Precision.HIGHEST,
                         preferred_element_type=jnp.float32)

BR = 2000
NB = N // BR

_row_spec = lambda w: pl.BlockSpec((BR, w), lambda i: (i, 0))
_full_spec = lambda r, w: pl.BlockSpec((r, w), lambda i: (0, 0))
_STATS = jax.ShapeDtypeStruct((2, H), jnp.float32)


def _accum_stats(i, vals, acc, stat_out):
  s1 = jnp.sum(vals, axis=0, keepdims=True)
  s2 = jnp.sum(vals * vals, axis=0, keepdims=True)
  part = jnp.concatenate([s1, s2], axis=0)

  @pl.when(i == 0)
  def _():
    acc[...] = part

  @pl.when(i > 0)
  def _():
    acc[...] = acc[...] + part

  @pl.when(i == NB - 1)
  def _():
    stat_out[...] = acc[...]


def _bn_from_stats(t, stats, gamma, beta):
  m = stats[0:1] * (1.0 / N)
  v = stats[1:2] * (1.0 / N) - m * m
  return (t - m) * lax.rsqrt(v + EPS) * gamma + beta


def _lin_body(x, agg, Wr, br, t_out, stat_out, acc):
  i = pl.program_id(0)
  t = _DOT(x[...] + agg[...], Wr[...]) + br[...]
  t_out[...] = t
  _accum_stats(i, t, acc, stat_out)


def _lin(x, agg, Wr, br, win):
  return pl.pallas_call(
      _lin_body,
      grid=(NB,),
      in_specs=[_row_spec(win), _row_spec(win), _full_spec(win, H),
                _full_spec(1, H)],
      out_specs=(_row_spec(H), _full_spec(2, H)),
      out_shape=(jax.ShapeDtypeStruct((N, H), jnp.float32), _STATS),
      scratch_shapes=[pltpu.VMEM((2, H), jnp.float32)],
  )(x, agg, Wr, br)


def _bnrelu_body(t, stats, g, bt, u_out, stat_out, acc):
  i = pl.program_id(0)
  u = jnp.maximum(_bn_from_stats(t[...], stats[...], g[...], bt[...]), 0.0)
  u_out[...] = u
  _accum_stats(i, u, acc, stat_out)


def _bnrelu(t, stats, g, bt):
  return pl.pallas_call(
      _bnrelu_body,
      grid=(NB,),
      in_specs=[_row_spec(H), _full_spec(2, H), _full_spec(1, H),
                _full_spec(1, H)],
      out_specs=(_row_spec(H), _full_spec(2, H)),
      out_shape=(jax.ShapeDtypeStruct((N, H), jnp.float32), _STATS),
      scratch_shapes=[pltpu.VMEM((2, H), jnp.float32)],
  )(t, stats, g, bt)


def _bnlin_body(u, stats, g, bt, Wr, br, out, stat_out, acc, *, relu_out):
  i = pl.program_id(0)
  z = _DOT(_bn_from_stats(u[...], stats[...], g[...], bt[...]), Wr[...]) + br[...]
  if relu_out:
    z = jnp.maximum(z, 0.0)
  out[...] = z
  _accum_stats(i, z, acc, stat_out)


def _bnlin(u, stats, g, bt, Wr, br, relu_out):
  return pl.pallas_call(
      functools.partial(_bnlin_body, relu_out=relu_out),
      grid=(NB,),
      in_specs=[_row_spec(H), _full_spec(2, H), _full_spec(1, H),
                _full_spec(1, H), _full_spec(H, H), _full_spec(1, H)],
      out_specs=(_row_spec(H), _full_spec(2, H)),
      out_shape=(jax.ShapeDtypeStruct((N, H), jnp.float32), _STATS),
      scratch_shapes=[pltpu.VMEM((2, H), jnp.float32)],
  )(u, stats, g, bt, Wr, br)


def _bnrelu_pool_body(t, stats, g, bt, batch, h2_out, pool_out,
                      acc_pool, acc_cnt):
  i = pl.program_id(0)
  h2 = jnp.maximum(_bn_from_stats(t[...], stats[...], g[...], bt[...]), 0.0)
  h2_out[...] = h2
  seg = lax.broadcasted_iota(jnp.int32, (BR, G), 1)
  onehot = (batch[...] == seg).astype(jnp.float32)
  part = lax.dot_general(onehot, h2, (((0,), (0,)), ((), ())),
                         precision=jax.lax.Precision.HIGHEST,
                         preferred_element_type=jnp.float32)
  cnt = jnp.sum(onehot, axis=0)[:, None]

  @pl.when(i == 0)
  def _():
    acc_pool[...] = part
    acc_cnt[...] = cnt

  @pl.when(i > 0)
  def _():
    acc_pool[...] = acc_pool[...] + part
    acc_cnt[...] = acc_cnt[...] + cnt

  @pl.when(i == NB - 1)
  def _():
    pool_out[...] = acc_pool[...] / jnp.maximum(acc_cnt[...], 1.0)


def _bnrelu_pool(t, stats, g, bt, batch):
  return pl.pallas_call(
      _bnrelu_pool_body,
      grid=(NB,),
      in_specs=[_row_spec(H), _full_spec(2, H), _full_spec(1, H),
                _full_spec(1, H), _row_spec(1)],
      out_specs=(_row_spec(H), _full_spec(G, H)),
      out_shape=(jax.ShapeDtypeStruct((N, H), jnp.float32),
                 jax.ShapeDtypeStruct((G, H), jnp.float32)),
      scratch_shapes=[pltpu.VMEM((G, H), jnp.float32),
                      pltpu.VMEM((G, 1), jnp.float32)],
  )(t, stats, g, bt, batch)


def kernel(x, edge_index, batch, W1, b1, W2, b2, W3, b3, W4, b4,
           g1, bt1, g2, bt2, g3, bt3, g4, bt4):
  ei = edge_index.astype(jnp.int32)
  src_p = jnp.concatenate(
      [ei[0], jnp.zeros((EPAD - E,), jnp.int32)]).reshape(NCHUNKS, CHUNK)
  dst_p = jnp.concatenate(
      [ei[1], jnp.full((EPAD - E,), N, jnp.int32)]).reshape(NCHUNKS, CHUNK)
  tile_id = lax.broadcasted_iota(jnp.int32, (NCHUNKS, CHUNK), 0) // CPT
  dummy = NHALF + tile_id
  dst0 = jnp.where(dst_p < NHALF, dst_p, dummy)
  dst1 = jnp.where((dst_p >= NHALF) & (dst_p < N), dst_p - NHALF, dummy)

  zeros = jnp.zeros((NACC, W), jnp.float32)
  a1lo, a1hi = _make_seg_l1()(x, src_p, dst0, dst1, zeros)
  agg1 = jnp.concatenate([a1lo[:NHALF], a1hi[:N - NHALF]], axis=0)

  t1, st1 = _lin(x, agg1, W1, b1.reshape(1, H), F)
  u, st2 = _bnrelu(t1, st1, g1.reshape(1, H), bt1.reshape(1, H))
  h, _ = _bnlin(u, st2, g2.reshape(1, H), bt2.reshape(1, H), W2,
                b2.reshape(1, H), relu_out=True)

  ha = h[:, :H // 2]
  hb = h[:, H // 2:]
  seg_l2 = _make_seg_l2()
  a2lo_a, a2lo_b = seg_l2(ha, hb, src_p, dst0, zeros)
  a2hi_a, a2hi_b = seg_l2(ha, hb, src_p, dst1, zeros)
  agg2 = jnp.concatenate(
      [jnp.concatenate([a2lo_a[:NHALF], a2lo_b[:NHALF]], axis=1),
       jnp.concatenate([a2hi_a[:N - NHALF], a2hi_b[:N - NHALF]], axis=1)],
      axis=0)

  batch2d = batch.astype(jnp.int32).reshape(N, 1)
  t2, st3 = _lin(h, agg2, W3, b3.reshape(1, H), H)
  v, st4 = _bnlin(t2, st3, g3.reshape(1, H), bt3.reshape(1, H), W4,
                  b4.reshape(1, H), relu_out=False)
  h2, x_pool = _bnrelu_pool(v, st4, g4.reshape(1, H), bt4.reshape(1, H),
                            batch2d)
  return (x_pool, h2)


# speedup vs baseline: 4.6339x; 2.5364x over previous
import functools

import jax
import jax.numpy as jnp
from jax import lax
from jax.experimental import pallas as pl
from jax.experimental.pallas import tpu as pltpu
from jax.experimental.pallas import tpu_sc as plsc

N = 10000
E = 320000
F = 128
H = 256
G = 64

NUM_CORES = 2
NUM_TILES = 16
W = 128
CHUNK = 128
CPT = 160
RAW = CPT * CHUNK
EPAD = NUM_TILES * RAW
NHALF = 5056
NACC = 5120
ROWS_PER_TILE = NACC // NUM_TILES
CAPR = RAW + 1024
RAWC = 2048
NSTAGE = RAW // RAWC
L = 16
EPS = 1e-5


def _zero_acc(zeros, acc, s):
  pltpu.sync_copy(zeros.at[pl.ds(s * ROWS_PER_TILE, ROWS_PER_TILE)],
                  acc.at[pl.ds(s * ROWS_PER_TILE, ROWS_PER_TILE)])


def _partition(srcs, dsts, s, src1d, dst1d, src_part, dst_part):
  dummy = NHALF + s

  def step(k, carry):
    off0, off1 = carry
    vs = src1d[pl.ds(k * L, L)]
    vd = dst1d[pl.ds(k * L, L)]
    m0 = vd < NHALF
    m1 = jnp.logical_and(vd >= NHALF, vd < N)
    c0 = plsc.all_reduce_population_count(m0)[0]
    c1 = plsc.all_reduce_population_count(m1)[0]
    plsc.store_compressed(src_part.at[pl.ds(off0, L)], vs, mask=m0)
    plsc.store_compressed(dst_part.at[pl.ds(off0, L)], vd, mask=m0)
    noff1 = off1 - c1
    plsc.store_compressed(src_part.at[pl.ds(noff1, L)], vs, mask=m1)
    plsc.store_compressed(dst_part.at[pl.ds(noff1, L)], vd - NHALF, mask=m1)
    return off0 + c0, noff1

  off0, off1 = jnp.int32(0), jnp.int32(CAPR)
  for t in range(NSTAGE):
    pltpu.sync_copy(srcs.at[pl.ds(s * RAW + t * RAWC, RAWC)], src1d)
    pltpu.sync_copy(dsts.at[pl.ds(s * RAW + t * RAWC, RAWC)], dst1d)
    off0, off1 = lax.fori_loop(0, RAWC // L, step, (off0, off1))
  n0 = off0
  n1 = CAPR - off1
  iota = lax.broadcasted_iota(jnp.int32, (L,), 0)

  base0 = (off0 // L) * L
  for k in range(17):
    o = base0 + k * L
    pos = o + iota
    cs = src_part[pl.ds(o, L)]
    cd = dst_part[pl.ds(o, L)]
    src_part[pl.ds(o, L)] = jnp.where(pos < n0, cs, 0)
    dst_part[pl.ds(o, L)] = jnp.where(pos < n0, cd, dummy)

  base1 = ((off1 - 256) // L) * L
  for k in range(17):
    o = base1 + k * L
    pos = o + iota
    cs = src_part[pl.ds(o, L)]
    cd = dst_part[pl.ds(o, L)]
    src_part[pl.ds(o, L)] = jnp.where(pos >= off1, cs, 0)
    dst_part[pl.ds(o, L)] = jnp.where(pos >= off1, cd, dummy)

  nch0 = (n0 + 255) // 256 * 2
  nch1 = (n1 + 255) // 256 * 2
  start1 = CAPR - nch1 * CHUNK
  return nch0, start1, nch1


def _process(table, src_part, dst_part, start, nch, rows, dstchunk, acc, sems):
  for b in range(2):
    @pl.when(b < nch)
    def _():
      pltpu.async_copy(
          table.at[src_part.at[pl.ds(start + b * CHUNK, CHUNK)]],
          rows[b], sems[b])

  @pl.loop(0, nch, step=2)
  def _(j):
    for b in range(2):
      jj = j + b
      e0 = start + jj * CHUNK
      pltpu.make_async_copy(
          table.at[src_part.at[pl.ds(e0, CHUNK)]], rows[b], sems[b]).wait()
      for k in range(CHUNK // L):
        dstchunk[b, pl.ds(k * L, L)] = dst_part[pl.ds(e0 + k * L, L)]
      pltpu.sync_copy(rows[b], acc.at[dstchunk.at[b]], add=True)

      @pl.when(jj + 2 < nch)
      def _():
        pltpu.async_copy(
            table.at[src_part.at[pl.ds(e0 + 2 * CHUNK, CHUNK)]],
            rows[b], sems[b])


def _copy_out(acc, out, s):
  pltpu.sync_copy(acc.at[pl.ds(s * ROWS_PER_TILE, ROWS_PER_TILE)],
                  out.at[pl.ds(s * ROWS_PER_TILE, ROWS_PER_TILE)])


_SC_SCRATCH = [
    pltpu.VMEM((RAWC,), jnp.int32),
    pltpu.VMEM((RAWC,), jnp.int32),
    pltpu.VMEM((CAPR,), jnp.int32),
    pltpu.VMEM((CAPR,), jnp.int32),
    pltpu.VMEM((CHUNK, W), jnp.float32),
    pltpu.VMEM((CHUNK, W), jnp.float32),
    pltpu.VMEM((2, CHUNK), jnp.int32),
    pltpu.VMEM_SHARED((NACC, W), jnp.float32),
    pltpu.SemaphoreType.DMA,
    pltpu.SemaphoreType.DMA,
]


def _stage_and_partition(srcs, dsts, src1d, dst1d, s, src_part, dst_part):
  return _partition(srcs, dsts, s, src1d, dst1d, src_part, dst_part)


@functools.cache
def _make_seg_l1():
  mesh = plsc.VectorSubcoreMesh(core_axis_name="c", subcore_axis_name="s")

  def body(table, srcs, dsts, zeros, out_a, out_b,
           src1d, dst1d, src_part, dst_part, rows0, rows1, dstchunk, acc,
           sem0, sem1):
    c = lax.axis_index("c")
    s = lax.axis_index("s")
    _zero_acc(zeros, acc, s)
    nch0, start1, nch1 = _stage_and_partition(
        srcs, dsts, src1d, dst1d, s, src_part, dst_part)
    plsc.subcore_barrier()

    @pl.when(c == 0)
    def _():
      _process(table, src_part, dst_part, 0, nch0,
               (rows0, rows1), dstchunk, acc, (sem0, sem1))

    @pl.when(c == 1)
    def _():
      _process(table, src_part, dst_part, start1, nch1,
               (rows0, rows1), dstchunk, acc, (sem0, sem1))

    plsc.subcore_barrier()

    @pl.when(c == 0)
    def _():
      _copy_out(acc, out_a, s)

    @pl.when(c == 1)
    def _():
      _copy_out(acc, out_b, s)

  return pl.kernel(
      body,
      out_type=(jax.ShapeDtypeStruct((NACC, W), jnp.float32),
                jax.ShapeDtypeStruct((NACC, W), jnp.float32)),
      mesh=mesh, scratch_types=_SC_SCRATCH,
      compiler_params=pltpu.CompilerParams(needs_layout_passes=False))


@functools.cache
def _make_seg_l2():
  mesh = plsc.VectorSubcoreMesh(core_axis_name="c", subcore_axis_name="s")

  def body(table_a, table_b, srcs, dsts, zeros, out0a, out0b, out1a, out1b,
           src1d, dst1d, src_part, dst_part, rows0, rows1, dstchunk, acc,
           sem0, sem1):
    c = lax.axis_index("c")
    s = lax.axis_index("s")
    _zero_acc(zeros, acc, s)
    nch0, start1, nch1 = _stage_and_partition(
        srcs, dsts, src1d, dst1d, s, src_part, dst_part)
    plsc.subcore_barrier()

    def phase(start, nch, oa, ob):
      @pl.when(c == 0)
      def _():
        _process(table_a, src_part, dst_part, start, nch,
                 (rows0, rows1), dstchunk, acc, (sem0, sem1))

      @pl.when(c == 1)
      def _():
        _process(table_b, src_part, dst_part, start, nch,
                 (rows0, rows1), dstchunk, acc, (sem0, sem1))

      plsc.subcore_barrier()

      @pl.when(c == 0)
      def _():
        _copy_out(acc, oa, s)

      @pl.when(c == 1)
      def _():
        _copy_out(acc, ob, s)

    phase(0, nch0, out0a, out0b)
    _zero_acc(zeros, acc, s)
    plsc.subcore_barrier()
    phase(start1, nch1, out1a, out1b)

  return pl.kernel(
      body,
      out_type=tuple(jax.ShapeDtypeStruct((NACC, W), jnp.float32)
                     for _ in range(4)),
      mesh=mesh, scratch_types=_SC_SCRATCH,
      compiler_params=pltpu.CompilerParams(needs_layout_passes=False))


_DOT = functools.partial(jnp.dot, precision=jax.lax.Precision.HIGHEST,
                         preferred_element_type=jnp.float32)

BR = 2000
NB = N // BR

_row_spec = lambda w: pl.BlockSpec((BR, w), lambda i: (i, 0))
_full_spec = lambda r, w: pl.BlockSpec((r, w), lambda i: (0, 0))
_STATS = jax.ShapeDtypeStruct((2, H), jnp.float32)


def _accum_stats(i, vals, acc, stat_out):
  s1 = jnp.sum(vals, axis=0, keepdims=True)
  s2 = jnp.sum(vals * vals, axis=0, keepdims=True)
  part = jnp.concatenate([s1, s2], axis=0)

  @pl.when(i == 0)
  def _():
    acc[...] = part

  @pl.when(i > 0)
  def _():
    acc[...] = acc[...] + part

  @pl.when(i == NB - 1)
  def _():
    stat_out[...] = acc[...]


def _bn_from_stats(t, stats, gamma, beta):
  m = stats[0:1] * (1.0 / N)
  v = stats[1:2] * (1.0 / N) - m * m
  return (t - m) * lax.rsqrt(v + EPS) * gamma + beta


def _lin_body(x, agg, Wr, br, t_out, stat_out, acc):
  i = pl.program_id(0)
  t = _DOT(x[...] + agg[...], Wr[...]) + br[...]
  t_out[...] = t
  _accum_stats(i, t, acc, stat_out)


def _lin(x, agg, Wr, br, win):
  return pl.pallas_call(
      _lin_body,
      grid=(NB,),
      in_specs=[_row_spec(win), _row_spec(win), _full_spec(win, H),
                _full_spec(1, H)],
      out_specs=(_row_spec(H), _full_spec(2, H)),
      out_shape=(jax.ShapeDtypeStruct((N, H), jnp.float32), _STATS),
      scratch_shapes=[pltpu.VMEM((2, H), jnp.float32)],
  )(x, agg, Wr, br)


def _bnrelu_body(t, stats, g, bt, u_out, stat_out, acc):
  i = pl.program_id(0)
  u = jnp.maximum(_bn_from_stats(t[...], stats[...], g[...], bt[...]), 0.0)
  u_out[...] = u
  _accum_stats(i, u, acc, stat_out)


def _bnrelu(t, stats, g, bt):
  return pl.pallas_call(
      _bnrelu_body,
      grid=(NB,),
      in_specs=[_row_spec(H), _full_spec(2, H), _full_spec(1, H),
                _full_spec(1, H)],
      out_specs=(_row_spec(H), _full_spec(2, H)),
      out_shape=(jax.ShapeDtypeStruct((N, H), jnp.float32), _STATS),
      scratch_shapes=[pltpu.VMEM((2, H), jnp.float32)],
  )(t, stats, g, bt)


def _bnlin_body(u, stats, g, bt, Wr, br, out, stat_out, acc, *, relu_out):
  i = pl.program_id(0)
  z = _DOT(_bn_from_stats(u[...], stats[...], g[...], bt[...]), Wr[...]) + br[...]
  if relu_out:
    z = jnp.maximum(z, 0.0)
  out[...] = z
  _accum_stats(i, z, acc, stat_out)


def _bnlin(u, stats, g, bt, Wr, br, relu_out):
  return pl.pallas_call(
      functools.partial(_bnlin_body, relu_out=relu_out),
      grid=(NB,),
      in_specs=[_row_spec(H), _full_spec(2, H), _full_spec(1, H),
                _full_spec(1, H), _full_spec(H, H), _full_spec(1, H)],
      out_specs=(_row_spec(H), _full_spec(2, H)),
      out_shape=(jax.ShapeDtypeStruct((N, H), jnp.float32), _STATS),
      scratch_shapes=[pltpu.VMEM((2, H), jnp.float32)],
  )(u, stats, g, bt, Wr, br)


def _bnrelu_pool_body(t, stats, g, bt, batch, h2_out, pool_out,
                      acc_pool, acc_cnt):
  i = pl.program_id(0)
  h2 = jnp.maximum(_bn_from_stats(t[...], stats[...], g[...], bt[...]), 0.0)
  h2_out[...] = h2
  seg = lax.broadcasted_iota(jnp.int32, (BR, G), 1)
  onehot = (batch[...] == seg).astype(jnp.float32)
  part = lax.dot_general(onehot, h2, (((0,), (0,)), ((), ())),
                         precision=jax.lax.Precision.HIGHEST,
                         preferred_element_type=jnp.float32)
  cnt = jnp.sum(onehot, axis=0)[:, None]

  @pl.when(i == 0)
  def _():
    acc_pool[...] = part
    acc_cnt[...] = cnt

  @pl.when(i > 0)
  def _():
    acc_pool[...] = acc_pool[...] + part
    acc_cnt[...] = acc_cnt[...] + cnt

  @pl.when(i == NB - 1)
  def _():
    pool_out[...] = acc_pool[...] / jnp.maximum(acc_cnt[...], 1.0)


def _bnrelu_pool(t, stats, g, bt, batch):
  return pl.pallas_call(
      _bnrelu_pool_body,
      grid=(NB,),
      in_specs=[_row_spec(H), _full_spec(2, H), _full_spec(1, H),
                _full_spec(1, H), _row_spec(1)],
      out_specs=(_row_spec(H), _full_spec(G, H)),
      out_shape=(jax.ShapeDtypeStruct((N, H), jnp.float32),
                 jax.ShapeDtypeStruct((G, H), jnp.float32)),
      scratch_shapes=[pltpu.VMEM((G, H), jnp.float32),
                      pltpu.VMEM((G, 1), jnp.float32)],
  )(t, stats, g, bt, batch)


def kernel(x, edge_index, batch, W1, b1, W2, b2, W3, b3, W4, b4,
           g1, bt1, g2, bt2, g3, bt3, g4, bt4):
  ei = edge_index.astype(jnp.int32)
  src_f = jnp.concatenate([ei[0], jnp.zeros((EPAD - E,), jnp.int32)])
  dst_f = jnp.concatenate([ei[1], jnp.full((EPAD - E,), N, jnp.int32)])

  zeros = jnp.zeros((NACC, W), jnp.float32)
  a1lo, a1hi = _make_seg_l1()(x, src_f, dst_f, zeros)
  agg1 = jnp.concatenate([a1lo[:NHALF], a1hi[:N - NHALF]], axis=0)

  t1, st1 = _lin(x, agg1, W1, b1.reshape(1, H), F)
  u, st2 = _bnrelu(t1, st1, g1.reshape(1, H), bt1.reshape(1, H))
  h, _ = _bnlin(u, st2, g2.reshape(1, H), bt2.reshape(1, H), W2,
                b2.reshape(1, H), relu_out=True)

  ha = h[:, :H // 2]
  hb = h[:, H // 2:]
  q0a, q0b, q1a, q1b = _make_seg_l2()(ha, hb, src_f, dst_f, zeros)
  agg2 = jnp.concatenate(
      [jnp.concatenate([q0a[:NHALF], q0b[:NHALF]], axis=1),
       jnp.concatenate([q1a[:N - NHALF], q1b[:N - NHALF]], axis=1)],
      axis=0)

  batch2d = batch.astype(jnp.int32).reshape(N, 1)
  t2, st3 = _lin(h, agg2, W3, b3.reshape(1, H), H)
  v, st4 = _bnlin(t2, st3, g3.reshape(1, H), bt3.reshape(1, H), W4,
                  b4.reshape(1, H), relu_out=False)
  h2, x_pool = _bnrelu_pool(v, st4, g4.reshape(1, H), bt4.reshape(1, H),
                            batch2d)
  return (x_pool, h2)


# speedup vs baseline: 4.7518x; 1.0255x over previous
import functools

import jax
import jax.numpy as jnp
from jax import lax
from jax.experimental import pallas as pl
from jax.experimental.pallas import tpu as pltpu
from jax.experimental.pallas import tpu_sc as plsc

N = 10000
E = 320000
F = 128
H = 256
G = 64

NUM_CORES = 2
NUM_TILES = 16
W = 128
CHUNK = 128
CPT = 160
RAW = CPT * CHUNK
EPAD = NUM_TILES * RAW
NHALF = 5056
NACC = 5120
ROWS_PER_TILE = NACC // NUM_TILES
NOUT = NHALF + NACC
CAPR = RAW + 1024
RAWC = 2048
NSTAGE = RAW // RAWC
L = 16
EPS = 1e-5


def _zero_acc(zeros, acc, s):
  pltpu.sync_copy(zeros.at[pl.ds(s * ROWS_PER_TILE, ROWS_PER_TILE)],
                  acc.at[pl.ds(s * ROWS_PER_TILE, ROWS_PER_TILE)])


def _partition(srcs, dsts, s, src1d, dst1d, src_part, dst_part):
  dummy = NHALF + s

  def step(k, carry):
    off0, off1 = carry
    vs = src1d[pl.ds(k * L, L)]
    vd = dst1d[pl.ds(k * L, L)]
    m0 = vd < NHALF
    m1 = jnp.logical_and(vd >= NHALF, vd < N)
    c0 = plsc.all_reduce_population_count(m0)[0]
    c1 = plsc.all_reduce_population_count(m1)[0]
    plsc.store_compressed(src_part.at[pl.ds(off0, L)], vs, mask=m0)
    plsc.store_compressed(dst_part.at[pl.ds(off0, L)], vd, mask=m0)
    noff1 = off1 - c1
    plsc.store_compressed(src_part.at[pl.ds(noff1, L)], vs, mask=m1)
    plsc.store_compressed(dst_part.at[pl.ds(noff1, L)], vd - NHALF, mask=m1)
    return off0 + c0, noff1

  off0, off1 = jnp.int32(0), jnp.int32(CAPR)
  for t in range(NSTAGE):
    pltpu.sync_copy(srcs.at[pl.ds(s * RAW + t * RAWC, RAWC)], src1d)
    pltpu.sync_copy(dsts.at[pl.ds(s * RAW + t * RAWC, RAWC)], dst1d)
    off0, off1 = lax.fori_loop(0, RAWC // L, step, (off0, off1))
  n0 = off0
  n1 = CAPR - off1
  iota = lax.broadcasted_iota(jnp.int32, (L,), 0)

  base0 = (off0 // L) * L
  for k in range(17):
    o = base0 + k * L
    pos = o + iota
    cs = src_part[pl.ds(o, L)]
    cd = dst_part[pl.ds(o, L)]
    src_part[pl.ds(o, L)] = jnp.where(pos < n0, cs, 0)
    dst_part[pl.ds(o, L)] = jnp.where(pos < n0, cd, dummy)

  base1 = ((off1 - 256) // L) * L
  for k in range(17):
    o = base1 + k * L
    pos = o + iota
    cs = src_part[pl.ds(o, L)]
    cd = dst_part[pl.ds(o, L)]
    src_part[pl.ds(o, L)] = jnp.where(pos >= off1, cs, 0)
    dst_part[pl.ds(o, L)] = jnp.where(pos >= off1, cd, dummy)

  nch0 = (n0 + 255) // 256 * 2
  nch1 = (n1 + 255) // 256 * 2
  start1 = CAPR - nch1 * CHUNK
  return nch0, start1, nch1


def _process(table, src_part, dst_part, start, nch, rows, dstchunk, acc, sems):
  for b in range(2):
    @pl.when(b < nch)
    def _():
      pltpu.async_copy(
          table.at[src_part.at[pl.ds(start + b * CHUNK, CHUNK)]],
          rows[b], sems[b])

  @pl.loop(0, nch, step=2)
  def _(j):
    for b in range(2):
      jj = j + b
      e0 = start + jj * CHUNK
      pltpu.make_async_copy(
          table.at[src_part.at[pl.ds(e0, CHUNK)]], rows[b], sems[b]).wait()
      for k in range(CHUNK // L):
        dstchunk[b, pl.ds(k * L, L)] = dst_part[pl.ds(e0 + k * L, L)]
      pltpu.sync_copy(rows[b], acc.at[dstchunk.at[b]], add=True)

      @pl.when(jj + 2 < nch)
      def _():
        pltpu.async_copy(
            table.at[src_part.at[pl.ds(e0 + 2 * CHUNK, CHUNK)]],
            rows[b], sems[b])


def _copy_out(acc, out, s, row_base):
  pltpu.sync_copy(acc.at[pl.ds(s * ROWS_PER_TILE, ROWS_PER_TILE)],
                  out.at[pl.ds(row_base + s * ROWS_PER_TILE, ROWS_PER_TILE)])


_SC_SCRATCH = [
    pltpu.VMEM((RAWC,), jnp.int32),
    pltpu.VMEM((RAWC,), jnp.int32),
    pltpu.VMEM((CAPR,), jnp.int32),
    pltpu.VMEM((CAPR,), jnp.int32),
    pltpu.VMEM((CHUNK, W), jnp.float32),
    pltpu.VMEM((CHUNK, W), jnp.float32),
    pltpu.VMEM((2, CHUNK), jnp.int32),
    pltpu.VMEM_SHARED((NACC, W), jnp.float32),
    pltpu.SemaphoreType.DMA,
    pltpu.SemaphoreType.DMA,
]


def _stage_and_partition(srcs, dsts, src1d, dst1d, s, src_part, dst_part):
  return _partition(srcs, dsts, s, src1d, dst1d, src_part, dst_part)


@functools.cache
def _make_seg_l1():
  mesh = plsc.VectorSubcoreMesh(core_axis_name="c", subcore_axis_name="s")

  def body(table, srcs, dsts, zeros, out,
           src1d, dst1d, src_part, dst_part, rows0, rows1, dstchunk, acc,
           sem0, sem1):
    c = lax.axis_index("c")
    s = lax.axis_index("s")
    _zero_acc(zeros, acc, s)
    nch0, start1, nch1 = _stage_and_partition(
        srcs, dsts, src1d, dst1d, s, src_part, dst_part)
    plsc.subcore_barrier()

    @pl.when(c == 0)
    def _():
      _process(table, src_part, dst_part, 0, nch0,
               (rows0, rows1), dstchunk, acc, (sem0, sem1))

    @pl.when(c == 1)
    def _():
      _process(table, src_part, dst_part, start1, nch1,
               (rows0, rows1), dstchunk, acc, (sem0, sem1))

    plsc.subcore_barrier()

    @pl.when(jnp.logical_and(c == 0, s < NUM_TILES - 1))
    def _():
      _copy_out(acc, out, s, 0)

    @pl.when(jnp.logical_and(c == 0, s == NUM_TILES - 1))
    def _():
      pltpu.sync_copy(
          acc.at[pl.ds(s * ROWS_PER_TILE, NHALF - s * ROWS_PER_TILE)],
          out.at[pl.ds(s * ROWS_PER_TILE, NHALF - s * ROWS_PER_TILE)])

    @pl.when(c == 1)
    def _():
      _copy_out(acc, out, s, NHALF)

  return pl.kernel(
      body,
      out_type=jax.ShapeDtypeStruct((NOUT, W), jnp.float32),
      mesh=mesh, scratch_types=_SC_SCRATCH,
      compiler_params=pltpu.CompilerParams(needs_layout_passes=False))


@functools.cache
def _make_seg_l2():
  mesh = plsc.VectorSubcoreMesh(core_axis_name="c", subcore_axis_name="s")

  def body(table_a, table_b, srcs, dsts, zeros, out_a, out_b,
           src1d, dst1d, src_part, dst_part, rows0, rows1, dstchunk, acc,
           sem0, sem1):
    c = lax.axis_index("c")
    s = lax.axis_index("s")
    _zero_acc(zeros, acc, s)
    nch0, start1, nch1 = _stage_and_partition(
        srcs, dsts, src1d, dst1d, s, src_part, dst_part)
    plsc.subcore_barrier()

    def phase(start, nch, row_base):
      @pl.when(c == 0)
      def _():
        _process(table_a, src_part, dst_part, start, nch,
                 (rows0, rows1), dstchunk, acc, (sem0, sem1))

      @pl.when(c == 1)
      def _():
        _process(table_b, src_part, dst_part, start, nch,
                 (rows0, rows1), dstchunk, acc, (sem0, sem1))

      plsc.subcore_barrier()

      @pl.when(c == 0)
      def _():
        _copy_out(acc, out_a, s, row_base)

      @pl.when(c == 1)
      def _():
        _copy_out(acc, out_b, s, row_base)

    phase(0, nch0, 0)
    _zero_acc(zeros, acc, s)
    plsc.subcore_barrier()
    phase(start1, nch1, NHALF)

  return pl.kernel(
      body,
      out_type=(jax.ShapeDtypeStruct((NOUT, W), jnp.float32),
                jax.ShapeDtypeStruct((NOUT, W), jnp.float32)),
      mesh=mesh, scratch_types=_SC_SCRATCH,
      compiler_params=pltpu.CompilerParams(needs_layout_passes=False))


_DOT = functools.partial(jnp.dot, precision=jax.lax.Precision.HIGHEST,
                         preferred_element_type=jnp.float32)

BR = 2000
NB = N // BR

_row_spec = lambda w: pl.BlockSpec((BR, w), lambda i: (i, 0))
_full_spec = lambda r, w: pl.BlockSpec((r, w), lambda i: (0, 0))
_STATS = jax.ShapeDtypeStruct((2, H), jnp.float32)


def _accum_stats(i, vals, acc, stat_out):
  s1 = jnp.sum(vals, axis=0, keepdims=True)
  s2 = jnp.sum(vals * vals, axis=0, keepdims=True)
  part = jnp.concatenate([s1, s2], axis=0)

  @pl.when(i == 0)
  def _():
    acc[...] = part

  @pl.when(i > 0)
  def _():
    acc[...] = acc[...] + part

  @pl.when(i == NB - 1)
  def _():
    stat_out[...] = acc[...]


def _bn_from_stats(t, stats, gamma, beta):
  m = stats[0:1] * (1.0 / N)
  v = stats[1:2] * (1.0 / N) - m * m
  return (t - m) * lax.rsqrt(v + EPS) * gamma + beta


def _lin_body(x, agg, Wr, br, t_out, stat_out, acc):
  i = pl.program_id(0)
  t = _DOT(x[...] + agg[...], Wr[...]) + br[...]
  t_out[...] = t
  _accum_stats(i, t, acc, stat_out)


def _lin(x, agg, Wr, br, win):
  return pl.pallas_call(
      _lin_body,
      grid=(NB,),
      in_specs=[_row_spec(win), _row_spec(win), _full_spec(win, H),
                _full_spec(1, H)],
      out_specs=(_row_spec(H), _full_spec(2, H)),
      out_shape=(jax.ShapeDtypeStruct((N, H), jnp.float32), _STATS),
      scratch_shapes=[pltpu.VMEM((2, H), jnp.float32)],
  )(x, agg, Wr, br)


def _lin2_body(ha, hb, qa, qb, Wa, Wb, br, t_out, stat_out, acc):
  i = pl.program_id(0)
  t = (_DOT(ha[...] + qa[...], Wa[...]) + _DOT(hb[...] + qb[...], Wb[...])
       + br[...])
  t_out[...] = t
  _accum_stats(i, t, acc, stat_out)


def _lin2(ha, hb, qa, qb, Wa, Wb, br):
  return pl.pallas_call(
      _lin2_body,
      grid=(NB,),
      in_specs=[_row_spec(W), _row_spec(W), _row_spec(W), _row_spec(W),
                _full_spec(W, H), _full_spec(W, H), _full_spec(1, H)],
      out_specs=(_row_spec(H), _full_spec(2, H)),
      out_shape=(jax.ShapeDtypeStruct((N, H), jnp.float32), _STATS),
      scratch_shapes=[pltpu.VMEM((2, H), jnp.float32)],
  )(ha, hb, qa, qb, Wa, Wb, br)


def _bnrelu_body(t, stats, g, bt, u_out, stat_out, acc):
  i = pl.program_id(0)
  u = jnp.maximum(_bn_from_stats(t[...], stats[...], g[...], bt[...]), 0.0)
  u_out[...] = u
  _accum_stats(i, u, acc, stat_out)


def _bnrelu(t, stats, g, bt):
  return pl.pallas_call(
      _bnrelu_body,
      grid=(NB,),
      in_specs=[_row_spec(H), _full_spec(2, H), _full_spec(1, H),
                _full_spec(1, H)],
      out_specs=(_row_spec(H), _full_spec(2, H)),
      out_shape=(jax.ShapeDtypeStruct((N, H), jnp.float32), _STATS),
      scratch_shapes=[pltpu.VMEM((2, H), jnp.float32)],
  )(t, stats, g, bt)


def _bnlin_body(u, stats, g, bt, Wr, br, out, stat_out, acc, *, relu_out):
  i = pl.program_id(0)
  z = _DOT(_bn_from_stats(u[...], stats[...], g[...], bt[...]), Wr[...]) + br[...]
  if relu_out:
    z = jnp.maximum(z, 0.0)
  out[...] = z
  _accum_stats(i, z, acc, stat_out)


def _bnlin(u, stats, g, bt, Wr, br, relu_out):
  return pl.pallas_call(
      functools.partial(_bnlin_body, relu_out=relu_out),
      grid=(NB,),
      in_specs=[_row_spec(H), _full_spec(2, H), _full_spec(1, H),
                _full_spec(1, H), _full_spec(H, H), _full_spec(1, H)],
      out_specs=(_row_spec(H), _full_spec(2, H)),
      out_shape=(jax.ShapeDtypeStruct((N, H), jnp.float32), _STATS),
      scratch_shapes=[pltpu.VMEM((2, H), jnp.float32)],
  )(u, stats, g, bt, Wr, br)


def _bnlinrelu_split_body(u, stats, g, bt, Wr, br, ha_out, hb_out):
  z = _DOT(_bn_from_stats(u[...], stats[...], g[...], bt[...]), Wr[...]) + br[...]
  z = jnp.maximum(z, 0.0)
  ha_out[...] = z[:, :W]
  hb_out[...] = z[:, W:]


def _bnlinrelu_split(u, stats, g, bt, Wr, br):
  return pl.pallas_call(
      _bnlinrelu_split_body,
      grid=(NB,),
      in_specs=[_row_spec(H), _full_spec(2, H), _full_spec(1, H),
                _full_spec(1, H), _full_spec(H, H), _full_spec(1, H)],
      out_specs=(_row_spec(W), _row_spec(W)),
      out_shape=(jax.ShapeDtypeStruct((N, W), jnp.float32),
                 jax.ShapeDtypeStruct((N, W), jnp.float32)),
  )(u, stats, g, bt, Wr, br)


def _bnrelu_pool_body(t, stats, g, bt, batch, h2_out, pool_out,
                      acc_pool, acc_cnt):
  i = pl.program_id(0)
  h2 = jnp.maximum(_bn_from_stats(t[...], stats[...], g[...], bt[...]), 0.0)
  h2_out[...] = h2
  seg = lax.broadcasted_iota(jnp.int32, (BR, G), 1)
  onehot = (batch[...] == seg).astype(jnp.float32)
  part = lax.dot_general(onehot, h2, (((0,), (0,)), ((), ())),
                         precision=jax.lax.Precision.HIGHEST,
                         preferred_element_type=jnp.float32)
  cnt = jnp.sum(onehot, axis=0)[:, None]

  @pl.when(i == 0)
  def _():
    acc_pool[...] = part
    acc_cnt[...] = cnt

  @pl.when(i > 0)
  def _():
    acc_pool[...] = acc_pool[...] + part
    acc_cnt[...] = acc_cnt[...] + cnt

  @pl.when(i == NB - 1)
  def _():
    pool_out[...] = acc_pool[...] / jnp.maximum(acc_cnt[...], 1.0)


def _bnrelu_pool(t, stats, g, bt, batch):
  return pl.pallas_call(
      _bnrelu_pool_body,
      grid=(NB,),
      in_specs=[_row_spec(H), _full_spec(2, H), _full_spec(1, H),
                _full_spec(1, H), _row_spec(1)],
      out_specs=(_row_spec(H), _full_spec(G, H)),
      out_shape=(jax.ShapeDtypeStruct((N, H), jnp.float32),
                 jax.ShapeDtypeStruct((G, H), jnp.float32)),
      scratch_shapes=[pltpu.VMEM((G, H), jnp.float32),
                      pltpu.VMEM((G, 1), jnp.float32)],
  )(t, stats, g, bt, batch)


def kernel(x, edge_index, batch, W1, b1, W2, b2, W3, b3, W4, b4,
           g1, bt1, g2, bt2, g3, bt3, g4, bt4):
  ei = edge_index.astype(jnp.int32)
  src_f = jnp.concatenate([ei[0], jnp.zeros((EPAD - E,), jnp.int32)])
  dst_f = jnp.concatenate([ei[1], jnp.full((EPAD - E,), N, jnp.int32)])

  zeros = jnp.zeros((NACC, W), jnp.float32)
  agg1 = _make_seg_l1()(x, src_f, dst_f, zeros)

  t1, st1 = _lin(x, agg1, W1, b1.reshape(1, H), F)
  u, st2 = _bnrelu(t1, st1, g1.reshape(1, H), bt1.reshape(1, H))
  ha, hb = _bnlinrelu_split(u, st2, g2.reshape(1, H), bt2.reshape(1, H), W2,
                            b2.reshape(1, H))

  qa, qb = _make_seg_l2()(ha, hb, src_f, dst_f, zeros)

  batch2d = batch.astype(jnp.int32).reshape(N, 1)
  t2, st3 = _lin2(ha, hb, qa, qb, W3[:W], W3[W:], b3.reshape(1, H))
  v, st4 = _bnlin(t2, st3, g3.reshape(1, H), bt3.reshape(1, H), W4,
                  b4.reshape(1, H), relu_out=False)
  h2, x_pool = _bnrelu_pool(v, st4, g4.reshape(1, H), bt4.reshape(1, H),
                            batch2d)
  return (x_pool, h2)


# speedup vs baseline: 4.9490x; 1.0415x over previous
import functools

import jax
import jax.numpy as jnp
from jax import lax
from jax.experimental import pallas as pl
from jax.experimental.pallas import tpu as pltpu
from jax.experimental.pallas import tpu_sc as plsc

N = 10000
E = 320000
F = 128
H = 256
G = 64

NUM_CORES = 2
NUM_TILES = 16
W = 128
CHUNK = 64
NBUF = 4
PADE = 256
RAW = 20480
EPAD = NUM_TILES * RAW
NHALF = 5056
NACC = 5120
ROWS_PER_TILE = NACC // NUM_TILES
NOUT = NHALF + NACC
CAPR = RAW + 1024
RAWC = 2048
NSTAGE = RAW // RAWC
L = 16
EPS = 1e-5


def _zero_acc(zeros, acc, s):
  pltpu.sync_copy(zeros.at[pl.ds(s * ROWS_PER_TILE, ROWS_PER_TILE)],
                  acc.at[pl.ds(s * ROWS_PER_TILE, ROWS_PER_TILE)])


def _partition(srcs, dsts, s, src1d, dst1d, src_part, dst_part):
  dummy = NHALF + s

  def step(k, carry):
    off0, off1 = carry
    vs = src1d[pl.ds(k * L, L)]
    vd = dst1d[pl.ds(k * L, L)]
    m0 = vd < NHALF
    m1 = jnp.logical_and(vd >= NHALF, vd < N)
    c0 = plsc.all_reduce_population_count(m0)[0]
    c1 = plsc.all_reduce_population_count(m1)[0]
    plsc.store_compressed(src_part.at[pl.ds(off0, L)], vs, mask=m0)
    plsc.store_compressed(dst_part.at[pl.ds(off0, L)], vd, mask=m0)
    noff1 = off1 - c1
    plsc.store_compressed(src_part.at[pl.ds(noff1, L)], vs, mask=m1)
    plsc.store_compressed(dst_part.at[pl.ds(noff1, L)], vd - NHALF, mask=m1)
    return off0 + c0, noff1

  off0, off1 = jnp.int32(0), jnp.int32(CAPR)
  for t in range(NSTAGE):
    pltpu.sync_copy(srcs.at[pl.ds(s * RAW + t * RAWC, RAWC)], src1d)
    pltpu.sync_copy(dsts.at[pl.ds(s * RAW + t * RAWC, RAWC)], dst1d)
    off0, off1 = lax.fori_loop(0, RAWC // L, step, (off0, off1))
  n0 = off0
  n1 = CAPR - off1
  iota = lax.broadcasted_iota(jnp.int32, (L,), 0)

  base0 = (off0 // L) * L
  for k in range(17):
    o = base0 + k * L
    pos = o + iota
    cs = src_part[pl.ds(o, L)]
    cd = dst_part[pl.ds(o, L)]
    src_part[pl.ds(o, L)] = jnp.where(pos < n0, cs, 0)
    dst_part[pl.ds(o, L)] = jnp.where(pos < n0, cd, dummy)

  base1 = ((off1 - 256) // L) * L
  for k in range(17):
    o = base1 + k * L
    pos = o + iota
    cs = src_part[pl.ds(o, L)]
    cd = dst_part[pl.ds(o, L)]
    src_part[pl.ds(o, L)] = jnp.where(pos >= off1, cs, 0)
    dst_part[pl.ds(o, L)] = jnp.where(pos >= off1, cd, dummy)

  nch0 = (n0 + PADE - 1) // PADE * (PADE // CHUNK)
  nch1 = (n1 + PADE - 1) // PADE * (PADE // CHUNK)
  start1 = CAPR - nch1 * CHUNK
  return nch0, start1, nch1


def _process(table, src_part, dst_part, start, nch, rows, dstchunk, acc, sems):
  for b in range(NBUF):
    @pl.when(b < nch)
    def _():
      pltpu.async_copy(
          table.at[src_part.at[pl.ds(start + b * CHUNK, CHUNK)]],
          rows[b], sems[b])

  @pl.loop(0, nch, step=NBUF)
  def _(j):
    for b in range(NBUF):
      jj = j + b
      e0 = start + jj * CHUNK
      pltpu.make_async_copy(
          table.at[src_part.at[pl.ds(e0, CHUNK)]], rows[b], sems[b]).wait()
      for k in range(CHUNK // L):
        dstchunk[b, pl.ds(k * L, L)] = dst_part[pl.ds(e0 + k * L, L)]
      pltpu.sync_copy(rows[b], acc.at[dstchunk.at[b]], add=True)

      @pl.when(jj + NBUF < nch)
      def _():
        pltpu.async_copy(
            table.at[src_part.at[pl.ds(e0 + NBUF * CHUNK, CHUNK)]],
            rows[b], sems[b])


def _copy_out(acc, out, s, row_base):
  pltpu.sync_copy(acc.at[pl.ds(s * ROWS_PER_TILE, ROWS_PER_TILE)],
                  out.at[pl.ds(row_base + s * ROWS_PER_TILE, ROWS_PER_TILE)])


_SC_SCRATCH = [
    pltpu.VMEM((RAWC,), jnp.int32),
    pltpu.VMEM((RAWC,), jnp.int32),
    pltpu.VMEM((CAPR,), jnp.int32),
    pltpu.VMEM((CAPR,), jnp.int32),
] + [pltpu.VMEM((CHUNK, W), jnp.float32) for _ in range(NBUF)] + [
    pltpu.VMEM((NBUF, CHUNK), jnp.int32),
    pltpu.VMEM_SHARED((NACC, W), jnp.float32),
] + [pltpu.SemaphoreType.DMA for _ in range(NBUF)]


def _stage_and_partition(srcs, dsts, src1d, dst1d, s, src_part, dst_part):
  return _partition(srcs, dsts, s, src1d, dst1d, src_part, dst_part)


@functools.cache
def _make_seg_l1():
  mesh = plsc.VectorSubcoreMesh(core_axis_name="c", subcore_axis_name="s")

  def body(table, srcs, dsts, zeros, out,
           src1d, dst1d, src_part, dst_part, r0, r1, r2, r3, dstchunk, acc,
           m0, m1, m2, m3):
    c = lax.axis_index("c")
    s = lax.axis_index("s")
    _zero_acc(zeros, acc, s)
    nch0, start1, nch1 = _stage_and_partition(
        srcs, dsts, src1d, dst1d, s, src_part, dst_part)
    plsc.subcore_barrier()

    @pl.when(c == 0)
    def _():
      _process(table, src_part, dst_part, 0, nch0,
               (r0, r1, r2, r3), dstchunk, acc, (m0, m1, m2, m3))

    @pl.when(c == 1)
    def _():
      _process(table, src_part, dst_part, start1, nch1,
               (r0, r1, r2, r3), dstchunk, acc, (m0, m1, m2, m3))

    plsc.subcore_barrier()

    @pl.when(jnp.logical_and(c == 0, s < NUM_TILES - 1))
    def _():
      _copy_out(acc, out, s, 0)

    @pl.when(jnp.logical_and(c == 0, s == NUM_TILES - 1))
    def _():
      pltpu.sync_copy(
          acc.at[pl.ds(s * ROWS_PER_TILE, NHALF - s * ROWS_PER_TILE)],
          out.at[pl.ds(s * ROWS_PER_TILE, NHALF - s * ROWS_PER_TILE)])

    @pl.when(c == 1)
    def _():
      _copy_out(acc, out, s, NHALF)

  return pl.kernel(
      body,
      out_type=jax.ShapeDtypeStruct((NOUT, W), jnp.float32),
      mesh=mesh, scratch_types=_SC_SCRATCH,
      compiler_params=pltpu.CompilerParams(needs_layout_passes=False))


@functools.cache
def _make_seg_l2():
  mesh = plsc.VectorSubcoreMesh(core_axis_name="c", subcore_axis_name="s")

  def body(table_a, table_b, srcs, dsts, zeros, out_a, out_b,
           src1d, dst1d, src_part, dst_part, r0, r1, r2, r3, dstchunk, acc,
           m0, m1, m2, m3):
    c = lax.axis_index("c")
    s = lax.axis_index("s")
    _zero_acc(zeros, acc, s)
    nch0, start1, nch1 = _stage_and_partition(
        srcs, dsts, src1d, dst1d, s, src_part, dst_part)
    plsc.subcore_barrier()

    def phase(start, nch, row_base):
      @pl.when(c == 0)
      def _():
        _process(table_a, src_part, dst_part, start, nch,
                 (r0, r1, r2, r3), dstchunk, acc, (m0, m1, m2, m3))

      @pl.when(c == 1)
      def _():
        _process(table_b, src_part, dst_part, start, nch,
                 (r0, r1, r2, r3), dstchunk, acc, (m0, m1, m2, m3))

      plsc.subcore_barrier()

      @pl.when(c == 0)
      def _():
        _copy_out(acc, out_a, s, row_base)

      @pl.when(c == 1)
      def _():
        _copy_out(acc, out_b, s, row_base)

    phase(0, nch0, 0)
    _zero_acc(zeros, acc, s)
    plsc.subcore_barrier()
    phase(start1, nch1, NHALF)

  return pl.kernel(
      body,
      out_type=(jax.ShapeDtypeStruct((NOUT, W), jnp.float32),
                jax.ShapeDtypeStruct((NOUT, W), jnp.float32)),
      mesh=mesh, scratch_types=_SC_SCRATCH,
      compiler_params=pltpu.CompilerParams(needs_layout_passes=False))


_DOT = functools.partial(jnp.dot, precision=jax.lax.Precision.HIGHEST,
                         preferred_element_type=jnp.float32)

BR = 2000
NB = N // BR

_row_spec = lambda w: pl.BlockSpec((BR, w), lambda i: (i, 0))
_full_spec = lambda r, w: pl.BlockSpec((r, w), lambda i: (0, 0))
_STATS = jax.ShapeDtypeStruct((2, H), jnp.float32)


def _accum_stats(i, vals, acc, stat_out):
  s1 = jnp.sum(vals, axis=0, keepdims=True)
  s2 = jnp.sum(vals * vals, axis=0, keepdims=True)
  part = jnp.concatenate([s1, s2], axis=0)

  @pl.when(i == 0)
  def _():
    acc[...] = part

  @pl.when(i > 0)
  def _():
    acc[...] = acc[...] + part

  @pl.when(i == NB - 1)
  def _():
    stat_out[...] = acc[...]


def _bn_from_stats(t, stats, gamma, beta):
  m = stats[0:1] * (1.0 / N)
  v = stats[1:2] * (1.0 / N) - m * m
  return (t - m) * lax.rsqrt(v + EPS) * gamma + beta


def _lin_body(x, agg, Wr, br, t_out, stat_out, acc):
  i = pl.program_id(0)
  t = _DOT(x[...] + agg[...], Wr[...]) + br[...]
  t_out[...] = t
  _accum_stats(i, t, acc, stat_out)


def _lin(x, agg, Wr, br, win):
  return pl.pallas_call(
      _lin_body,
      grid=(NB,),
      in_specs=[_row_spec(win), _row_spec(win), _full_spec(win, H),
                _full_spec(1, H)],
      out_specs=(_row_spec(H), _full_spec(2, H)),
      out_shape=(jax.ShapeDtypeStruct((N, H), jnp.float32), _STATS),
      scratch_shapes=[pltpu.VMEM((2, H), jnp.float32)],
  )(x, agg, Wr, br)


def _lin2_body(ha, hb, qa, qb, Wa, Wb, br, t_out, stat_out, acc):
  i = pl.program_id(0)
  t = (_DOT(ha[...] + qa[...], Wa[...]) + _DOT(hb[...] + qb[...], Wb[...])
       + br[...])
  t_out[...] = t
  _accum_stats(i, t, acc, stat_out)


def _lin2(ha, hb, qa, qb, Wa, Wb, br):
  return pl.pallas_call(
      _lin2_body,
      grid=(NB,),
      in_specs=[_row_spec(W), _row_spec(W), _row_spec(W), _row_spec(W),
                _full_spec(W, H), _full_spec(W, H), _full_spec(1, H)],
      out_specs=(_row_spec(H), _full_spec(2, H)),
      out_shape=(jax.ShapeDtypeStruct((N, H), jnp.float32), _STATS),
      scratch_shapes=[pltpu.VMEM((2, H), jnp.float32)],
  )(ha, hb, qa, qb, Wa, Wb, br)


def _bnrelu_body(t, stats, g, bt, u_out, stat_out, acc):
  i = pl.program_id(0)
  u = jnp.maximum(_bn_from_stats(t[...], stats[...], g[...], bt[...]), 0.0)
  u_out[...] = u
  _accum_stats(i, u, acc, stat_out)


def _bnrelu(t, stats, g, bt):
  return pl.pallas_call(
      _bnrelu_body,
      grid=(NB,),
      in_specs=[_row_spec(H), _full_spec(2, H), _full_spec(1, H),
                _full_spec(1, H)],
      out_specs=(_row_spec(H), _full_spec(2, H)),
      out_shape=(jax.ShapeDtypeStruct((N, H), jnp.float32), _STATS),
      scratch_shapes=[pltpu.VMEM((2, H), jnp.float32)],
  )(t, stats, g, bt)


def _bnlin_body(u, stats, g, bt, Wr, br, out, stat_out, acc, *, relu_out):
  i = pl.program_id(0)
  z = _DOT(_bn_from_stats(u[...], stats[...], g[...], bt[...]), Wr[...]) + br[...]
  if relu_out:
    z = jnp.maximum(z, 0.0)
  out[...] = z
  _accum_stats(i, z, acc, stat_out)


def _bnlin(u, stats, g, bt, Wr, br, relu_out):
  return pl.pallas_call(
      functools.partial(_bnlin_body, relu_out=relu_out),
      grid=(NB,),
      in_specs=[_row_spec(H), _full_spec(2, H), _full_spec(1, H),
                _full_spec(1, H), _full_spec(H, H), _full_spec(1, H)],
      out_specs=(_row_spec(H), _full_spec(2, H)),
      out_shape=(jax.ShapeDtypeStruct((N, H), jnp.float32), _STATS),
      scratch_shapes=[pltpu.VMEM((2, H), jnp.float32)],
  )(u, stats, g, bt, Wr, br)


def _bnlinrelu_split_body(u, stats, g, bt, Wr, br, ha_out, hb_out):
  z = _DOT(_bn_from_stats(u[...], stats[...], g[...], bt[...]), Wr[...]) + br[...]
  z = jnp.maximum(z, 0.0)
  ha_out[...] = z[:, :W]
  hb_out[...] = z[:, W:]


def _bnlinrelu_split(u, stats, g, bt, Wr, br):
  return pl.pallas_call(
      _bnlinrelu_split_body,
      grid=(NB,),
      in_specs=[_row_spec(H), _full_spec(2, H), _full_spec(1, H),
                _full_spec(1, H), _full_spec(H, H), _full_spec(1, H)],
      out_specs=(_row_spec(W), _row_spec(W)),
      out_shape=(jax.ShapeDtypeStruct((N, W), jnp.float32),
                 jax.ShapeDtypeStruct((N, W), jnp.float32)),
  )(u, stats, g, bt, Wr, br)


def _bnrelu_pool_body(t, stats, g, bt, batch, h2_out, pool_out,
                      acc_pool, acc_cnt):
  i = pl.program_id(0)
  h2 = jnp.maximum(_bn_from_stats(t[...], stats[...], g[...], bt[...]), 0.0)
  h2_out[...] = h2
  seg = lax.broadcasted_iota(jnp.int32, (BR, G), 1)
  onehot = (batch[...] == seg).astype(jnp.float32)
  part = lax.dot_general(onehot, h2, (((0,), (0,)), ((), ())),
                         precision=jax.lax.Precision.HIGHEST,
                         preferred_element_type=jnp.float32)
  cnt = jnp.sum(onehot, axis=0)[:, None]

  @pl.when(i == 0)
  def _():
    acc_pool[...] = part
    acc_cnt[...] = cnt

  @pl.when(i > 0)
  def _():
    acc_pool[...] = acc_pool[...] + part
    acc_cnt[...] = acc_cnt[...] + cnt

  @pl.when(i == NB - 1)
  def _():
    pool_out[...] = acc_pool[...] / jnp.maximum(acc_cnt[...], 1.0)


def _bnrelu_pool(t, stats, g, bt, batch):
  return pl.pallas_call(
      _bnrelu_pool_body,
      grid=(NB,),
      in_specs=[_row_spec(H), _full_spec(2, H), _full_spec(1, H),
                _full_spec(1, H), _row_spec(1)],
      out_specs=(_row_spec(H), _full_spec(G, H)),
      out_shape=(jax.ShapeDtypeStruct((N, H), jnp.float32),
                 jax.ShapeDtypeStruct((G, H), jnp.float32)),
      scratch_shapes=[pltpu.VMEM((G, H), jnp.float32),
                      pltpu.VMEM((G, 1), jnp.float32)],
  )(t, stats, g, bt, batch)


def kernel(x, edge_index, batch, W1, b1, W2, b2, W3, b3, W4, b4,
           g1, bt1, g2, bt2, g3, bt3, g4, bt4):
  ei = edge_index.astype(jnp.int32)
  src_f = jnp.concatenate([ei[0], jnp.zeros((EPAD - E,), jnp.int32)])
  dst_f = jnp.concatenate([ei[1], jnp.full((EPAD - E,), N, jnp.int32)])

  zeros = jnp.zeros((NACC, W), jnp.float32)
  agg1 = _make_seg_l1()(x, src_f, dst_f, zeros)

  t1, st1 = _lin(x, agg1, W1, b1.reshape(1, H), F)
  u, st2 = _bnrelu(t1, st1, g1.reshape(1, H), bt1.reshape(1, H))
  ha, hb = _bnlinrelu_split(u, st2, g2.reshape(1, H), bt2.reshape(1, H), W2,
                            b2.reshape(1, H))

  qa, qb = _make_seg_l2()(ha, hb, src_f, dst_f, zeros)

  batch2d = batch.astype(jnp.int32).reshape(N, 1)
  t2, st3 = _lin2(ha, hb, qa, qb, W3[:W], W3[W:], b3.reshape(1, H))
  v, st4 = _bnlin(t2, st3, g3.reshape(1, H), bt3.reshape(1, H), W4,
                  b4.reshape(1, H), relu_out=False)
  h2, x_pool = _bnrelu_pool(v, st4, g4.reshape(1, H), bt4.reshape(1, H),
                            batch2d)
  return (x_pool, h2)


# speedup vs baseline: 5.1382x; 1.0382x over previous
import functools

import jax
import jax.numpy as jnp
from jax import lax
from jax.experimental import pallas as pl
from jax.experimental.pallas import tpu as pltpu
from jax.experimental.pallas import tpu_sc as plsc

N = 10000
E = 320000
F = 128
H = 256
G = 64

NUM_CORES = 2
NUM_TILES = 16
W = 128
CHUNK = 64
NBUF = 4
PADE = 256
RAW = 20480
EPAD = NUM_TILES * RAW
NHALF = 5056
NACC = 5120
ROWS_PER_TILE = NACC // NUM_TILES
NOUT = NHALF + NACC
CAPR = RAW + 1024
RAWC = 2048
NSTAGE = RAW // RAWC
L = 16
EPS = 1e-5


def _zero_acc(zeros, acc, s):
  pltpu.sync_copy(zeros.at[pl.ds(s * ROWS_PER_TILE, ROWS_PER_TILE)],
                  acc.at[pl.ds(s * ROWS_PER_TILE, ROWS_PER_TILE)])


def _partition(srcs, dsts, s, src1d, dst1d, src_part, dst_part):
  dummy = NHALF + s

  U = 4

  def step(k, carry):
    off0, off1 = carry
    vss, vds, ms0, ms1, cs0, cs1 = [], [], [], [], [], []
    for i in range(U):
      vs = src1d[pl.ds((k * U + i) * L, L)]
      vd = dst1d[pl.ds((k * U + i) * L, L)]
      m0 = vd < NHALF
      m1 = jnp.logical_and(vd >= NHALF, vd < N)
      vss.append(vs)
      vds.append(vd)
      ms0.append(m0)
      ms1.append(m1)
      cs0.append(plsc.all_reduce_population_count(m0)[0])
      cs1.append(plsc.all_reduce_population_count(m1)[0])
    for i in range(U):
      plsc.store_compressed(src_part.at[pl.ds(off0, L)], vss[i], mask=ms0[i])
      plsc.store_compressed(dst_part.at[pl.ds(off0, L)], vds[i], mask=ms0[i])
      off0 = off0 + cs0[i]
      off1 = off1 - cs1[i]
      plsc.store_compressed(src_part.at[pl.ds(off1, L)], vss[i], mask=ms1[i])
      plsc.store_compressed(dst_part.at[pl.ds(off1, L)], vds[i] - NHALF,
                            mask=ms1[i])
    return off0, off1

  off0, off1 = jnp.int32(0), jnp.int32(CAPR)
  for t in range(NSTAGE):
    pltpu.sync_copy(srcs.at[pl.ds(s * RAW + t * RAWC, RAWC)], src1d)
    pltpu.sync_copy(dsts.at[pl.ds(s * RAW + t * RAWC, RAWC)], dst1d)
    off0, off1 = lax.fori_loop(0, RAWC // (L * U), step, (off0, off1))
  n0 = off0
  n1 = CAPR - off1
  iota = lax.broadcasted_iota(jnp.int32, (L,), 0)

  base0 = (off0 // L) * L
  for k in range(17):
    o = base0 + k * L
    pos = o + iota
    cs = src_part[pl.ds(o, L)]
    cd = dst_part[pl.ds(o, L)]
    src_part[pl.ds(o, L)] = jnp.where(pos < n0, cs, 0)
    dst_part[pl.ds(o, L)] = jnp.where(pos < n0, cd, dummy)

  base1 = ((off1 - 256) // L) * L
  for k in range(17):
    o = base1 + k * L
    pos = o + iota
    cs = src_part[pl.ds(o, L)]
    cd = dst_part[pl.ds(o, L)]
    src_part[pl.ds(o, L)] = jnp.where(pos >= off1, cs, 0)
    dst_part[pl.ds(o, L)] = jnp.where(pos >= off1, cd, dummy)

  nch0 = (n0 + PADE - 1) // PADE * (PADE // CHUNK)
  nch1 = (n1 + PADE - 1) // PADE * (PADE // CHUNK)
  start1 = CAPR - nch1 * CHUNK
  return nch0, start1, nch1


def _process(table, src_part, dst_part, start, nch, rows, dstchunk, acc, sems):
  for b in range(NBUF):
    @pl.when(b < nch)
    def _():
      pltpu.async_copy(
          table.at[src_part.at[pl.ds(start + b * CHUNK, CHUNK)]],
          rows[b], sems[b])

  @pl.loop(0, nch, step=NBUF)
  def _(j):
    for b in range(NBUF):
      jj = j + b
      e0 = start + jj * CHUNK
      pltpu.make_async_copy(
          table.at[src_part.at[pl.ds(e0, CHUNK)]], rows[b], sems[b]).wait()
      for k in range(CHUNK // L):
        dstchunk[b, pl.ds(k * L, L)] = dst_part[pl.ds(e0 + k * L, L)]
      pltpu.sync_copy(rows[b], acc.at[dstchunk.at[b]], add=True)

      @pl.when(jj + NBUF < nch)
      def _():
        pltpu.async_copy(
            table.at[src_part.at[pl.ds(e0 + NBUF * CHUNK, CHUNK)]],
            rows[b], sems[b])


def _copy_out(acc, out, s, row_base):
  pltpu.sync_copy(acc.at[pl.ds(s * ROWS_PER_TILE, ROWS_PER_TILE)],
                  out.at[pl.ds(row_base + s * ROWS_PER_TILE, ROWS_PER_TILE)])


_SC_SCRATCH = [
    pltpu.VMEM((RAWC,), jnp.int32),
    pltpu.VMEM((RAWC,), jnp.int32),
    pltpu.VMEM((CAPR,), jnp.int32),
    pltpu.VMEM((CAPR,), jnp.int32),
] + [pltpu.VMEM((CHUNK, W), jnp.float32) for _ in range(NBUF)] + [
    pltpu.VMEM((NBUF, CHUNK), jnp.int32),
    pltpu.VMEM_SHARED((NACC, W), jnp.float32),
] + [pltpu.SemaphoreType.DMA for _ in range(NBUF)]


def _stage_and_partition(srcs, dsts, src1d, dst1d, s, src_part, dst_part):
  return _partition(srcs, dsts, s, src1d, dst1d, src_part, dst_part)


@functools.cache
def _make_seg_l1():
  mesh = plsc.VectorSubcoreMesh(core_axis_name="c", subcore_axis_name="s")

  def body(table, srcs, dsts, zeros, out,
           src1d, dst1d, src_part, dst_part, r0, r1, r2, r3, dstchunk, acc,
           m0, m1, m2, m3):
    c = lax.axis_index("c")
    s = lax.axis_index("s")
    _zero_acc(zeros, acc, s)
    nch0, start1, nch1 = _stage_and_partition(
        srcs, dsts, src1d, dst1d, s, src_part, dst_part)
    plsc.subcore_barrier()

    @pl.when(c == 0)
    def _():
      _process(table, src_part, dst_part, 0, nch0,
               (r0, r1, r2, r3), dstchunk, acc, (m0, m1, m2, m3))

    @pl.when(c == 1)
    def _():
      _process(table, src_part, dst_part, start1, nch1,
               (r0, r1, r2, r3), dstchunk, acc, (m0, m1, m2, m3))

    plsc.subcore_barrier()

    @pl.when(jnp.logical_and(c == 0, s < NUM_TILES - 1))
    def _():
      _copy_out(acc, out, s, 0)

    @pl.when(jnp.logical_and(c == 0, s == NUM_TILES - 1))
    def _():
      pltpu.sync_copy(
          acc.at[pl.ds(s * ROWS_PER_TILE, NHALF - s * ROWS_PER_TILE)],
          out.at[pl.ds(s * ROWS_PER_TILE, NHALF - s * ROWS_PER_TILE)])

    @pl.when(c == 1)
    def _():
      _copy_out(acc, out, s, NHALF)

  return pl.kernel(
      body,
      out_type=jax.ShapeDtypeStruct((NOUT, W), jnp.float32),
      mesh=mesh, scratch_types=_SC_SCRATCH,
      compiler_params=pltpu.CompilerParams(needs_layout_passes=False))


@functools.cache
def _make_seg_l2():
  mesh = plsc.VectorSubcoreMesh(core_axis_name="c", subcore_axis_name="s")

  def body(table_a, table_b, srcs, dsts, zeros, out_a, out_b,
           src1d, dst1d, src_part, dst_part, r0, r1, r2, r3, dstchunk, acc,
           m0, m1, m2, m3):
    c = lax.axis_index("c")
    s = lax.axis_index("s")
    _zero_acc(zeros, acc, s)
    nch0, start1, nch1 = _stage_and_partition(
        srcs, dsts, src1d, dst1d, s, src_part, dst_part)
    plsc.subcore_barrier()

    def phase(start, nch, row_base):
      @pl.when(c == 0)
      def _():
        _process(table_a, src_part, dst_part, start, nch,
                 (r0, r1, r2, r3), dstchunk, acc, (m0, m1, m2, m3))

      @pl.when(c == 1)
      def _():
        _process(table_b, src_part, dst_part, start, nch,
                 (r0, r1, r2, r3), dstchunk, acc, (m0, m1, m2, m3))

      plsc.subcore_barrier()

      @pl.when(c == 0)
      def _():
        _copy_out(acc, out_a, s, row_base)

      @pl.when(c == 1)
      def _():
        _copy_out(acc, out_b, s, row_base)

    phase(0, nch0, 0)
    _zero_acc(zeros, acc, s)
    plsc.subcore_barrier()
    phase(start1, nch1, NHALF)

  return pl.kernel(
      body,
      out_type=(jax.ShapeDtypeStruct((NOUT, W), jnp.float32),
                jax.ShapeDtypeStruct((NOUT, W), jnp.float32)),
      mesh=mesh, scratch_types=_SC_SCRATCH,
      compiler_params=pltpu.CompilerParams(needs_layout_passes=False))


_DOT = functools.partial(jnp.dot, precision=jax.lax.Precision.HIGHEST,
                         preferred_element_type=jnp.float32)

BR = 2000
NB = N // BR

_row_spec = lambda w: pl.BlockSpec((BR, w), lambda i: (i, 0))
_full_spec = lambda r, w: pl.BlockSpec((r, w), lambda i: (0, 0))
_STATS = jax.ShapeDtypeStruct((2, H), jnp.float32)


def _accum_stats(i, vals, acc, stat_out):
  s1 = jnp.sum(vals, axis=0, keepdims=True)
  s2 = jnp.sum(vals * vals, axis=0, keepdims=True)
  part = jnp.concatenate([s1, s2], axis=0)

  @pl.when(i == 0)
  def _():
    acc[...] = part

  @pl.when(i > 0)
  def _():
    acc[...] = acc[...] + part

  @pl.when(i == NB - 1)
  def _():
    stat_out[...] = acc[...]


def _bn_from_stats(t, stats, gamma, beta):
  m = stats[0:1] * (1.0 / N)
  v = stats[1:2] * (1.0 / N) - m * m
  return (t - m) * lax.rsqrt(v + EPS) * gamma + beta


def _lin_body(x, agg, Wr, br, t_out, stat_out, acc):
  i = pl.program_id(0)
  t = _DOT(x[...] + agg[...], Wr[...]) + br[...]
  t_out[...] = t
  _accum_stats(i, t, acc, stat_out)


def _lin(x, agg, Wr, br, win):
  return pl.pallas_call(
      _lin_body,
      grid=(NB,),
      in_specs=[_row_spec(win), _row_spec(win), _full_spec(win, H),
                _full_spec(1, H)],
      out_specs=(_row_spec(H), _full_spec(2, H)),
      out_shape=(jax.ShapeDtypeStruct((N, H), jnp.float32), _STATS),
      scratch_shapes=[pltpu.VMEM((2, H), jnp.float32)],
  )(x, agg, Wr, br)


def _lin2_body(ha, hb, qa, qb, Wa, Wb, br, t_out, stat_out, acc):
  i = pl.program_id(0)
  t = (_DOT(ha[...] + qa[...], Wa[...]) + _DOT(hb[...] + qb[...], Wb[...])
       + br[...])
  t_out[...] = t
  _accum_stats(i, t, acc, stat_out)


def _lin2(ha, hb, qa, qb, Wa, Wb, br):
  return pl.pallas_call(
      _lin2_body,
      grid=(NB,),
      in_specs=[_row_spec(W), _row_spec(W), _row_spec(W), _row_spec(W),
                _full_spec(W, H), _full_spec(W, H), _full_spec(1, H)],
      out_specs=(_row_spec(H), _full_spec(2, H)),
      out_shape=(jax.ShapeDtypeStruct((N, H), jnp.float32), _STATS),
      scratch_shapes=[pltpu.VMEM((2, H), jnp.float32)],
  )(ha, hb, qa, qb, Wa, Wb, br)


def _bnrelu_body(t, stats, g, bt, u_out, stat_out, acc):
  i = pl.program_id(0)
  u = jnp.maximum(_bn_from_stats(t[...], stats[...], g[...], bt[...]), 0.0)
  u_out[...] = u
  _accum_stats(i, u, acc, stat_out)


def _bnrelu(t, stats, g, bt):
  return pl.pallas_call(
      _bnrelu_body,
      grid=(NB,),
      in_specs=[_row_spec(H), _full_spec(2, H), _full_spec(1, H),
                _full_spec(1, H)],
      out_specs=(_row_spec(H), _full_spec(2, H)),
      out_shape=(jax.ShapeDtypeStruct((N, H), jnp.float32), _STATS),
      scratch_shapes=[pltpu.VMEM((2, H), jnp.float32)],
  )(t, stats, g, bt)


def _bnlin_body(u, stats, g, bt, Wr, br, out, stat_out, acc, *, relu_out):
  i = pl.program_id(0)
  z = _DOT(_bn_from_stats(u[...], stats[...], g[...], bt[...]), Wr[...]) + br[...]
  if relu_out:
    z = jnp.maximum(z, 0.0)
  out[...] = z
  _accum_stats(i, z, acc, stat_out)


def _bnlin(u, stats, g, bt, Wr, br, relu_out):
  return pl.pallas_call(
      functools.partial(_bnlin_body, relu_out=relu_out),
      grid=(NB,),
      in_specs=[_row_spec(H), _full_spec(2, H), _full_spec(1, H),
                _full_spec(1, H), _full_spec(H, H), _full_spec(1, H)],
      out_specs=(_row_spec(H), _full_spec(2, H)),
      out_shape=(jax.ShapeDtypeStruct((N, H), jnp.float32), _STATS),
      scratch_shapes=[pltpu.VMEM((2, H), jnp.float32)],
  )(u, stats, g, bt, Wr, br)


def _bnlinrelu_split_body(u, stats, g, bt, Wr, br, ha_out, hb_out):
  z = _DOT(_bn_from_stats(u[...], stats[...], g[...], bt[...]), Wr[...]) + br[...]
  z = jnp.maximum(z, 0.0)
  ha_out[...] = z[:, :W]
  hb_out[...] = z[:, W:]


def _bnlinrelu_split(u, stats, g, bt, Wr, br):
  return pl.pallas_call(
      _bnlinrelu_split_body,
      grid=(NB,),
      in_specs=[_row_spec(H), _full_spec(2, H), _full_spec(1, H),
                _full_spec(1, H), _full_spec(H, H), _full_spec(1, H)],
      out_specs=(_row_spec(W), _row_spec(W)),
      out_shape=(jax.ShapeDtypeStruct((N, W), jnp.float32),
                 jax.ShapeDtypeStruct((N, W), jnp.float32)),
  )(u, stats, g, bt, Wr, br)


def _bnrelu_pool_body(t, stats, g, bt, batch, h2_out, pool_out,
                      acc_pool, acc_cnt):
  i = pl.program_id(0)
  h2 = jnp.maximum(_bn_from_stats(t[...], stats[...], g[...], bt[...]), 0.0)
  h2_out[...] = h2
  seg = lax.broadcasted_iota(jnp.int32, (BR, G), 1)
  onehot = (batch[...] == seg).astype(jnp.float32)
  part = lax.dot_general(onehot, h2, (((0,), (0,)), ((), ())),
                         precision=jax.lax.Precision.HIGHEST,
                         preferred_element_type=jnp.float32)
  cnt = jnp.sum(onehot, axis=0)[:, None]

  @pl.when(i == 0)
  def _():
    acc_pool[...] = part
    acc_cnt[...] = cnt

  @pl.when(i > 0)
  def _():
    acc_pool[...] = acc_pool[...] + part
    acc_cnt[...] = acc_cnt[...] + cnt

  @pl.when(i == NB - 1)
  def _():
    pool_out[...] = acc_pool[...] / jnp.maximum(acc_cnt[...], 1.0)


def _bnrelu_pool(t, stats, g, bt, batch):
  return pl.pallas_call(
      _bnrelu_pool_body,
      grid=(NB,),
      in_specs=[_row_spec(H), _full_spec(2, H), _full_spec(1, H),
                _full_spec(1, H), _row_spec(1)],
      out_specs=(_row_spec(H), _full_spec(G, H)),
      out_shape=(jax.ShapeDtypeStruct((N, H), jnp.float32),
                 jax.ShapeDtypeStruct((G, H), jnp.float32)),
      scratch_shapes=[pltpu.VMEM((G, H), jnp.float32),
                      pltpu.VMEM((G, 1), jnp.float32)],
  )(t, stats, g, bt, batch)


def kernel(x, edge_index, batch, W1, b1, W2, b2, W3, b3, W4, b4,
           g1, bt1, g2, bt2, g3, bt3, g4, bt4):
  ei = edge_index.astype(jnp.int32)
  src_f = jnp.concatenate([ei[0], jnp.zeros((EPAD - E,), jnp.int32)])
  dst_f = jnp.concatenate([ei[1], jnp.full((EPAD - E,), N, jnp.int32)])

  zeros = jnp.zeros((NACC, W), jnp.float32)
  agg1 = _make_seg_l1()(x, src_f, dst_f, zeros)

  t1, st1 = _lin(x, agg1, W1, b1.reshape(1, H), F)
  u, st2 = _bnrelu(t1, st1, g1.reshape(1, H), bt1.reshape(1, H))
  ha, hb = _bnlinrelu_split(u, st2, g2.reshape(1, H), bt2.reshape(1, H), W2,
                            b2.reshape(1, H))

  qa, qb = _make_seg_l2()(ha, hb, src_f, dst_f, zeros)

  batch2d = batch.astype(jnp.int32).reshape(N, 1)
  t2, st3 = _lin2(ha, hb, qa, qb, W3[:W], W3[W:], b3.reshape(1, H))
  v, st4 = _bnlin(t2, st3, g3.reshape(1, H), bt3.reshape(1, H), W4,
                  b4.reshape(1, H), relu_out=False)
  h2, x_pool = _bnrelu_pool(v, st4, g4.reshape(1, H), bt4.reshape(1, H),
                            batch2d)
  return (x_pool, h2)
